# baseline (device time: 197174 ns/iter reference)
import jax
import jax.numpy as jnp
from jax import lax
from jax.experimental import pallas as pl
from jax.experimental.pallas import tpu as pltpu

N_DEV = 8
B = 2
SEQ = 4096
TAPS = 4
COUT = 1024
N_CHUNK = 8
ROWS = B * SEQ
CH_ROWS = ROWS // N_CHUNK
HALF = CH_ROWS // 2
NSUBQ = 4
QROWS = HALF // NSUBQ
STEPS = N_DEV - 1
SUB = HALF
HALO = 8
SEQ_PER_CHUNK = 4
AG_SLOTS = 3


def kernel(x, k, Wp):
    cin = x.shape[2]

    def body(x_ref, k_ref, w_ref, out_ref,
             stage_ref, copy_sems, rs_recv, ag_recv,
             rs_send_sems, ag_send_sems, rs_sems, ag_sems, credit_sems):
        def perm(p):
            return jnp.where(p < 4, p, 11 - p)

        pos = lax.axis_index("i")
        my = perm(pos)
        left = perm(lax.rem(my + N_DEV - 1, N_DEV))
        right = perm(lax.rem(my + 1, N_DEV))

        barrier = pltpu.get_barrier_semaphore()
        for nbr in (left, right):
            pl.semaphore_signal(barrier, inc=1, device_id=(nbr,),
                                device_id_type=pl.DeviceIdType.MESH)
        pl.semaphore_wait(barrier, 2)

        kv = k_ref[:, :]
        w16 = w_ref[:, :].astype(jnp.bfloat16)

        def start_load(g, h, slot):
            b = lax.div(g, SEQ_PER_CHUNK)
            s0 = pl.multiple_of(
                lax.rem(g, SEQ_PER_CHUNK) * CH_ROWS + h * SUB, SUB)
            main = pltpu.make_async_copy(
                x_ref.at[b, pl.ds(s0, SUB), :],
                stage_ref.at[slot, pl.ds(HALO, SUB), :],
                copy_sems.at[slot],
            )
            halo = pltpu.make_async_copy(
                x_ref.at[b, pl.ds(
                    pl.multiple_of(jnp.maximum(s0 - HALO, 0), HALO), HALO), :],
                stage_ref.at[slot, pl.ds(0, HALO), :],
                copy_sems.at[slot],
            )
            main.start()
            halo.start()
            return main, halo, s0

        def finish_load(ld, slot):
            main, halo, s0 = ld
            main.wait()
            halo.wait()

            @pl.when(s0 == 0)
            def _():
                stage_ref[slot, 0:HALO, :] = jnp.zeros((HALO, cin), jnp.float32)

        def compute_half(slot):
            xp = stage_ref[slot]
            y = (xp[HALO:HALO + SUB] * kv[3]
                 + xp[HALO - 1:HALO - 1 + SUB] * kv[2]
                 + xp[HALO - 2:HALO - 2 + SUB] * kv[1]
                 + xp[HALO - 3:HALO - 3 + SUB] * kv[0])
            a = y * jax.nn.sigmoid(y)
            return jnp.dot(a.astype(jnp.bfloat16), w16,
                           preferred_element_type=jnp.float32
                           ).astype(jnp.bfloat16)

        def rows(d, sub):
            return pl.ds(d * HALF + sub * QROWS, QROWS)

        def rs_desc(d, sub, slot, src):
            return pltpu.make_async_remote_copy(
                src_ref=src,
                dst_ref=rs_recv.at[d, sub, slot],
                send_sem=rs_send_sems.at[d, sub],
                recv_sem=rs_sems.at[d, sub, slot],
                device_id=(right if d == 0 else left,),
                device_id_type=pl.DeviceIdType.MESH,
            )

        def ag_desc(d, sub, slot, src):
            return pltpu.make_async_remote_copy(
                src_ref=src,
                dst_ref=ag_recv.at[d, sub, slot],
                send_sem=ag_send_sems.at[d, sub],
                recv_sem=ag_sems.at[d, sub, slot],
                device_id=(right if d == 0 else left,),
                device_id_type=pl.DeviceIdType.MESH,
            )

        def credit_wait(phase):
            pl.semaphore_wait(credit_sems.at[phase, 0], 1)
            pl.semaphore_wait(credit_sems.at[phase, 1], 1)

        def credit_signal(phase):
            pl.semaphore_signal(credit_sems.at[phase, 0], inc=1,
                                device_id=(left,),
                                device_id_type=pl.DeviceIdType.MESH)
            pl.semaphore_signal(credit_sems.at[phase, 1], inc=1,
                                device_id=(right,),
                                device_id_type=pl.DeviceIdType.MESH)

        ld0 = start_load(my, 0, 0)
        ld1 = start_load(my, 1, 1)
        finish_load(ld0, 0)
        out_ref[my, pl.ds(0, HALF), :] = compute_half(0)
        for sub in range(NSUBQ):
            rs_desc(0, sub, 0, out_ref.at[my, rows(0, sub), :]).start()
        finish_load(ld1, 1)
        out_ref[my, pl.ds(HALF, HALF), :] = compute_half(1)
        for sub in range(NSUBQ):
            rs_desc(1, sub, 0, out_ref.at[my, rows(1, sub), :]).start()
        rc0 = lax.rem(my - 1 + N_DEV, N_DEV)
        rc1 = lax.rem(my + 1, N_DEV)
        ld0 = start_load(rc0, 0, 0)
        ld1 = start_load(rc1, 1, 1)
        finish_load(ld0, 0)
        out_ref[rc0, pl.ds(0, HALF), :] = compute_half(0)
        finish_load(ld1, 1)
        out_ref[rc1, pl.ds(HALF, HALF), :] = compute_half(1)

        for s in range(STEPS):
            slot = s % 2
            rc = (lax.rem(my - s - 1 + N_DEV, N_DEV),
                  lax.rem(my + s + 1, N_DEV))
            if 1 <= s <= STEPS - 2:
                credit_wait(0)
            for sub in range(NSUBQ):
                for d in (0, 1):
                    desc = rs_desc(d, sub, slot,
                                   out_ref.at[rc[d], rows(d, sub), :])
                    desc.wait()
                    acc = (out_ref[rc[d], rows(d, sub), :].astype(jnp.float32)
                           + rs_recv[d, sub, slot].astype(jnp.float32))
                    out_ref[rc[d], rows(d, sub), :] = acc.astype(jnp.bfloat16)
                    if s < STEPS - 1:
                        rs_desc(d, sub, (s + 1) % 2,
                                out_ref.at[rc[d], rows(d, sub), :]).start()
            if s <= STEPS - 3:
                credit_signal(0)
            if s < STEPS - 1:
                nc0 = lax.rem(my - s - 2 + N_DEV, N_DEV)
                nc1 = lax.rem(my + s + 2, N_DEV)
                ld0 = start_load(nc0, 0, 0)
                ld1 = start_load(nc1, 1, 1)
                finish_load(ld0, 0)
                out_ref[nc0, pl.ds(0, HALF), :] = compute_half(0)
                finish_load(ld1, 1)
                out_ref[nc1, pl.ds(HALF, HALF), :] = compute_half(1)

        oc = (rc1, rc0)
        for d in (0, 1):
            for sub in range(NSUBQ):
                ag_desc(d, sub, 0, out_ref.at[oc[d], rows(d, sub), :]).start()
        for s in range(STEPS):
            slot = s % AG_SLOTS
            rcg = (lax.rem(my - s + N_DEV, N_DEV), lax.rem(my + s, N_DEV))
            if 2 <= s <= STEPS - 2:
                credit_wait(1)
            for sub in range(NSUBQ):
                for d in (0, 1):
                    src = (out_ref.at[oc[d], rows(d, sub), :] if s == 0
                           else ag_recv.at[d, sub, (s - 1) % AG_SLOTS])
                    desc = ag_desc(d, sub, slot, src)
                    desc.wait()
                    if s < STEPS - 1:
                        ag_desc(d, sub, (s + 1) % AG_SLOTS,
                                ag_recv.at[d, sub, slot]).start()
                    out_ref[rcg[d], rows(d, sub), :] = ag_recv[d, sub, slot]
            if 1 <= s <= STEPS - 3:
                credit_signal(1)

    out = pl.pallas_call(
        body,
        out_shape=jax.ShapeDtypeStruct((N_CHUNK, CH_ROWS, COUT), jnp.bfloat16),
        in_specs=[
            pl.BlockSpec(memory_space=pltpu.MemorySpace.HBM),
            pl.BlockSpec(memory_space=pltpu.VMEM),
            pl.BlockSpec(memory_space=pltpu.VMEM),
        ],
        out_specs=pl.BlockSpec(memory_space=pltpu.VMEM),
        scratch_shapes=[
            pltpu.VMEM((2, SUB + HALO, cin), jnp.float32),
            pltpu.SemaphoreType.DMA((2,)),
            pltpu.VMEM((2, NSUBQ, 2, QROWS, COUT), jnp.bfloat16),
            pltpu.VMEM((2, NSUBQ, AG_SLOTS, QROWS, COUT), jnp.bfloat16),
            pltpu.SemaphoreType.DMA((2, NSUBQ)),
            pltpu.SemaphoreType.DMA((2, NSUBQ)),
            pltpu.SemaphoreType.DMA((2, NSUBQ, 2)),
            pltpu.SemaphoreType.DMA((2, NSUBQ, AG_SLOTS)),
            pltpu.SemaphoreType.REGULAR((2, 2)),
        ],
        compiler_params=pltpu.CompilerParams(
            collective_id=0, vmem_limit_bytes=63 * 1024 * 1024),
    )(x, k, Wp)
    return out.reshape(B, SEQ, COUT).astype(jnp.float32)


# device time: 196963 ns/iter; 1.0011x vs baseline; 1.0011x over previous
import jax
import jax.numpy as jnp
from jax import lax
from jax.experimental import pallas as pl
from jax.experimental.pallas import tpu as pltpu

N_DEV = 8
B = 2
SEQ = 4096
TAPS = 4
COUT = 1024
N_CHUNK = 8
ROWS = B * SEQ
CH_ROWS = ROWS // N_CHUNK
HALF = CH_ROWS // 2
QROWS = HALF // 2
STEPS = N_DEV - 1
SUB = HALF
HALO = 8
SEQ_PER_CHUNK = 4
AG_SLOTS = 3


def kernel(x, k, Wp):
    cin = x.shape[2]

    def body(x_ref, k_ref, w_ref, out_ref,
             stage_ref, copy_sems, rs_recv, ag_recv,
             rs_send_sems, ag_send_sems, rs_sems, ag_sems, credit_sems):
        def perm(p):
            return jnp.where(p < 4, p, 11 - p)

        pos = lax.axis_index("i")
        my = perm(pos)
        left = perm(lax.rem(my + N_DEV - 1, N_DEV))
        right = perm(lax.rem(my + 1, N_DEV))

        barrier = pltpu.get_barrier_semaphore()
        for nbr in (left, right):
            pl.semaphore_signal(barrier, inc=1, device_id=(nbr,),
                                device_id_type=pl.DeviceIdType.MESH)
        pl.semaphore_wait(barrier, 2)

        kv = k_ref[:, :]
        w16 = w_ref[:, :].astype(jnp.bfloat16)

        def start_load(g, h, slot):
            b = lax.div(g, SEQ_PER_CHUNK)
            s0 = pl.multiple_of(
                lax.rem(g, SEQ_PER_CHUNK) * CH_ROWS + h * SUB, SUB)
            main = pltpu.make_async_copy(
                x_ref.at[b, pl.ds(s0, SUB), :],
                stage_ref.at[slot, pl.ds(HALO, SUB), :],
                copy_sems.at[slot],
            )
            halo = pltpu.make_async_copy(
                x_ref.at[b, pl.ds(
                    pl.multiple_of(jnp.maximum(s0 - HALO, 0), HALO), HALO), :],
                stage_ref.at[slot, pl.ds(0, HALO), :],
                copy_sems.at[slot],
            )
            main.start()
            halo.start()
            return main, halo, s0

        def finish_load(ld, slot):
            main, halo, s0 = ld
            main.wait()
            halo.wait()

            @pl.when(s0 == 0)
            def _():
                stage_ref[slot, 0:HALO, :] = jnp.zeros((HALO, cin), jnp.float32)

        def compute_half(slot):
            xp = stage_ref[slot]
            y = (xp[HALO:HALO + SUB] * kv[3]
                 + xp[HALO - 1:HALO - 1 + SUB] * kv[2]
                 + xp[HALO - 2:HALO - 2 + SUB] * kv[1]
                 + xp[HALO - 3:HALO - 3 + SUB] * kv[0])
            a = y * jax.nn.sigmoid(y)
            return jnp.dot(a.astype(jnp.bfloat16), w16,
                           preferred_element_type=jnp.float32
                           ).astype(jnp.bfloat16)

        def rows(d, sub):
            return pl.ds(d * HALF + sub * QROWS, QROWS)

        def rs_desc(d, sub, slot, src):
            return pltpu.make_async_remote_copy(
                src_ref=src,
                dst_ref=rs_recv.at[d, sub, slot],
                send_sem=rs_send_sems.at[d, sub],
                recv_sem=rs_sems.at[d, sub, slot],
                device_id=(right if d == 0 else left,),
                device_id_type=pl.DeviceIdType.MESH,
            )

        def ag_desc(d, sub, slot, src):
            return pltpu.make_async_remote_copy(
                src_ref=src,
                dst_ref=ag_recv.at[d, sub, slot],
                send_sem=ag_send_sems.at[d, sub],
                recv_sem=ag_sems.at[d, sub, slot],
                device_id=(right if d == 0 else left,),
                device_id_type=pl.DeviceIdType.MESH,
            )

        def credit_wait(phase):
            pl.semaphore_wait(credit_sems.at[phase, 0], 1)
            pl.semaphore_wait(credit_sems.at[phase, 1], 1)

        def credit_signal(phase):
            pl.semaphore_signal(credit_sems.at[phase, 0], inc=1,
                                device_id=(left,),
                                device_id_type=pl.DeviceIdType.MESH)
            pl.semaphore_signal(credit_sems.at[phase, 1], inc=1,
                                device_id=(right,),
                                device_id_type=pl.DeviceIdType.MESH)

        ld0 = start_load(my, 0, 0)
        ld1 = start_load(my, 1, 1)
        finish_load(ld0, 0)
        out_ref[my, pl.ds(0, HALF), :] = compute_half(0)
        rs_desc(0, 0, 0, out_ref.at[my, rows(0, 0), :]).start()
        rs_desc(0, 1, 0, out_ref.at[my, rows(0, 1), :]).start()
        finish_load(ld1, 1)
        out_ref[my, pl.ds(HALF, HALF), :] = compute_half(1)
        rs_desc(1, 0, 0, out_ref.at[my, rows(1, 0), :]).start()
        rs_desc(1, 1, 0, out_ref.at[my, rows(1, 1), :]).start()
        rc0 = lax.rem(my - 1 + N_DEV, N_DEV)
        rc1 = lax.rem(my + 1, N_DEV)
        ld0 = start_load(rc0, 0, 0)
        ld1 = start_load(rc1, 1, 1)
        finish_load(ld0, 0)
        out_ref[rc0, pl.ds(0, HALF), :] = compute_half(0)
        finish_load(ld1, 1)
        out_ref[rc1, pl.ds(HALF, HALF), :] = compute_half(1)

        for s in range(STEPS):
            slot = s % 2
            rc = (lax.rem(my - s - 1 + N_DEV, N_DEV),
                  lax.rem(my + s + 1, N_DEV))
            if 1 <= s <= STEPS - 2:
                credit_wait(0)
            for sub in (0, 1):
                for d in (0, 1):
                    desc = rs_desc(d, sub, slot,
                                   out_ref.at[rc[d], rows(d, sub), :])
                    desc.wait()
                    acc = (out_ref[rc[d], rows(d, sub), :].astype(jnp.float32)
                           + rs_recv[d, sub, slot].astype(jnp.float32))
                    out_ref[rc[d], rows(d, sub), :] = acc.astype(jnp.bfloat16)
                    if s < STEPS - 1:
                        rs_desc(d, sub, (s + 1) % 2,
                                out_ref.at[rc[d], rows(d, sub), :]).start()
            if s <= STEPS - 3:
                credit_signal(0)
            if s < STEPS - 1:
                nc0 = lax.rem(my - s - 2 + N_DEV, N_DEV)
                nc1 = lax.rem(my + s + 2, N_DEV)
                ld0 = start_load(nc0, 0, 0)
                ld1 = start_load(nc1, 1, 1)
                finish_load(ld0, 0)
                out_ref[nc0, pl.ds(0, HALF), :] = compute_half(0)
                finish_load(ld1, 1)
                out_ref[nc1, pl.ds(HALF, HALF), :] = compute_half(1)

        oc = (rc1, rc0)
        for d in (0, 1):
            for sub in (0, 1):
                ag_desc(d, sub, 0, out_ref.at[oc[d], rows(d, sub), :]).start()
        for s in range(STEPS):
            slot = s % AG_SLOTS
            rcg = (lax.rem(my - s + N_DEV, N_DEV), lax.rem(my + s, N_DEV))
            if 2 <= s <= STEPS - 2:
                credit_wait(1)
            for sub in (0, 1):
                for d in (0, 1):
                    src = (out_ref.at[oc[d], rows(d, sub), :] if s == 0
                           else ag_recv.at[d, sub, (s - 1) % AG_SLOTS])
                    desc = ag_desc(d, sub, slot, src)
                    desc.wait()
                    if s < STEPS - 1:
                        ag_desc(d, sub, (s + 1) % AG_SLOTS,
                                ag_recv.at[d, sub, slot]).start()
                    out_ref[rcg[d], rows(d, sub), :] = ag_recv[d, sub, slot]
            if 1 <= s <= STEPS - 3:
                credit_signal(1)

    out = pl.pallas_call(
        body,
        out_shape=jax.ShapeDtypeStruct((N_CHUNK, CH_ROWS, COUT), jnp.bfloat16),
        in_specs=[
            pl.BlockSpec(memory_space=pltpu.MemorySpace.HBM),
            pl.BlockSpec(memory_space=pltpu.VMEM),
            pl.BlockSpec(memory_space=pltpu.VMEM),
        ],
        out_specs=pl.BlockSpec(memory_space=pltpu.VMEM),
        scratch_shapes=[
            pltpu.VMEM((2, SUB + HALO, cin), jnp.float32),
            pltpu.SemaphoreType.DMA((2,)),
            pltpu.VMEM((2, 2, 2, QROWS, COUT), jnp.bfloat16),
            pltpu.VMEM((2, 2, AG_SLOTS, QROWS, COUT), jnp.bfloat16),
            pltpu.SemaphoreType.DMA((2, 2)),
            pltpu.SemaphoreType.DMA((2, 2)),
            pltpu.SemaphoreType.DMA((2, 2, 2)),
            pltpu.SemaphoreType.DMA((2, 2, AG_SLOTS)),
            pltpu.SemaphoreType.REGULAR((2, 2)),
        ],
        compiler_params=pltpu.CompilerParams(
            collective_id=0, vmem_limit_bytes=63 * 1024 * 1024),
    )(x, k, Wp)
    return out.reshape(B, SEQ, COUT).astype(jnp.float32)
